# baseline (speedup 1.0000x reference)
import functools

import jax
import jax.numpy as jnp
from jax.experimental import pallas as pl
from jax.experimental.pallas import tpu as pltpu

N = 2048
H = 128
W = 128
M0 = 0.01
BOUND = M0 * N
K = 21
BIG = 3.4e38
TR = 4


def _dtm_kernel(x_ref, out_ref, d2_ref):
    i = pl.program_id(0)
    row = (i * TR + jax.lax.broadcasted_iota(jnp.int32, (1, TR, 1), 1)
           ).astype(jnp.float32)
    gy = 1.0 - row * (2.0 / (W - 1))
    gx = -1.0 + jax.lax.broadcasted_iota(
        jnp.int32, (1, 1, W), 2).astype(jnp.float32) * (2.0 / (W - 1))

    px = x_ref[:, 0:1].reshape(N, 1, 1)
    py = x_ref[:, 1:2].reshape(N, 1, 1)

    dx = px - gx
    dy = py - gy
    d2_ref[...] = dx * dx + dy * dy

    def body(_, carry):
        s, t, rem = carry
        d = d2_ref[...]
        m = jnp.min(d, axis=0, keepdims=True)
        mask = d == m
        c = jnp.sum(mask.astype(jnp.float32), axis=0, keepdims=True)
        d2_ref[...] = jnp.where(mask, BIG, d)
        take = jnp.minimum(c, rem)
        s = s + take * m
        t = jnp.where((rem > 0.0) & (rem <= take), m, t)
        rem = rem - take
        return s, t, rem

    zero = jnp.zeros((1, TR, W), jnp.float32)
    s, t, _ = jax.lax.fori_loop(
        0, K, body, (zero, zero, jnp.full((1, TR, W), float(K), jnp.float32)))

    dtm_val = s + t * (BOUND - K)
    out_ref[...] = jnp.sqrt(dtm_val / BOUND)


@jax.jit
def kernel(x):
    out = pl.pallas_call(
        _dtm_kernel,
        grid=(H // TR,),
        in_specs=[pl.BlockSpec((N, 2), lambda i: (0, 0))],
        out_specs=pl.BlockSpec((1, TR, W), lambda i: (i, 0, 0)),
        out_shape=jax.ShapeDtypeStruct((H // TR, TR, W), jnp.float32),
        scratch_shapes=[pltpu.VMEM((N, TR, W), jnp.float32)],
    )(x)
    return out.reshape(H, W)


# speedup vs baseline: 2.3970x; 2.3970x over previous
import functools

import jax
import jax.numpy as jnp
from jax.experimental import pallas as pl
from jax.experimental.pallas import tpu as pltpu

N = 2048
H = 128
W = 128
M0 = 0.01
BOUND = M0 * N
K = 21
BIG = 3.4e38


def _dtm_kernel(x_ref, out_ref, key_ref):
    i = pl.program_id(0)
    gy = 1.0 - i.astype(jnp.float32) * (2.0 / (W - 1))
    gx = -1.0 + jax.lax.broadcasted_iota(
        jnp.int32, (1, W), 1).astype(jnp.float32) * (2.0 / (W - 1))

    px = x_ref[:, 0:1]
    py = x_ref[:, 1:2]

    dx = px - gx
    dy = py - gy
    d2 = dx * dx + dy * dy

    row = jax.lax.broadcasted_iota(jnp.int32, (N, 1), 0)
    bits = jax.lax.bitcast_convert_type(d2, jnp.int32)
    key_ref[...] = jax.lax.bitcast_convert_type(
        (bits & jnp.int32(~2047)) | row, jnp.float32)

    def body(_, carry):
        s, v = carry
        d = key_ref[...]
        v = jnp.min(jnp.where(d > v, d, BIG), axis=0, keepdims=True)
        return s + v, v

    zero = jnp.zeros((1, W), jnp.float32)
    s, t = jax.lax.fori_loop(
        0, K, body, (zero, jnp.full((1, W), -1.0, jnp.float32)))

    dtm_val = s + t * (BOUND - K)
    out_ref[0] = jnp.sqrt(dtm_val / BOUND)


@jax.jit
def kernel(x):
    out = pl.pallas_call(
        _dtm_kernel,
        grid=(H,),
        in_specs=[pl.BlockSpec((N, 2), lambda i: (0, 0))],
        out_specs=pl.BlockSpec((1, 1, W), lambda i: (i, 0, 0)),
        out_shape=jax.ShapeDtypeStruct((H, 1, W), jnp.float32),
        scratch_shapes=[pltpu.VMEM((N, W), jnp.float32)],
    )(x)
    return out.reshape(H, W)


# speedup vs baseline: 4.2371x; 1.7677x over previous
import functools

import jax
import jax.numpy as jnp
from jax.experimental import pallas as pl
from jax.experimental.pallas import tpu as pltpu

N = 2048
H = 128
W = 128
M0 = 0.01
BOUND = M0 * N
K = 21
def _dtm_kernel(x_ref, out_ref, key_ref):
    MININT = jnp.int32(-2147483648)
    i = pl.program_id(0)
    gy = 1.0 - i.astype(jnp.float32) * (2.0 / (W - 1))
    gx = -1.0 + jax.lax.broadcasted_iota(
        jnp.int32, (1, W), 1).astype(jnp.float32) * (2.0 / (W - 1))

    px = x_ref[:, 0:1]
    py = x_ref[:, 1:2]

    dx = px - gx
    dy = py - gy
    d2 = dx * dx + dy * dy

    row = jax.lax.broadcasted_iota(jnp.int32, (N, 1), 0)
    bits = jax.lax.bitcast_convert_type(d2, jnp.int32)
    key_ref[...] = (bits & jnp.int32(~2047)) | row

    def body(_, carry):
        s, v = carry
        e = key_ref[...] - (v + (jnp.int32(1) - MININT))
        while e.shape[0] > 8:
            h = e.shape[0] // 2
            e = jnp.minimum(e[:h], e[h:])
        m = jnp.min(e, axis=0, keepdims=True)
        v = v + jnp.int32(1) + (m ^ MININT)
        s = s + jax.lax.bitcast_convert_type(v, jnp.float32)
        return s, v

    s, t = jax.lax.fori_loop(
        0, K, body,
        (jnp.zeros((1, W), jnp.float32), jnp.full((1, W), -1, jnp.int32)))

    tf = jax.lax.bitcast_convert_type(t, jnp.float32)
    dtm_val = s + tf * (BOUND - K)
    out_ref[0] = jnp.sqrt(dtm_val / BOUND)


@jax.jit
def kernel(x):
    out = pl.pallas_call(
        _dtm_kernel,
        grid=(H,),
        in_specs=[pl.BlockSpec((N, 2), lambda i: (0, 0))],
        out_specs=pl.BlockSpec((1, 1, W), lambda i: (i, 0, 0)),
        out_shape=jax.ShapeDtypeStruct((H, 1, W), jnp.float32),
        scratch_shapes=[pltpu.VMEM((N, W), jnp.int32)],
    )(x)
    return out.reshape(H, W)


# speedup vs baseline: 4.6922x; 1.1074x over previous
import functools

import jax
import jax.numpy as jnp
from jax.experimental import pallas as pl
from jax.experimental.pallas import tpu as pltpu

N = 2048
H = 128
W = 128
M0 = 0.01
BOUND = M0 * N
K = 21


def _dtm_kernel(x_ref, out_ref, key0_ref, key1_ref, dx2_ref):
    MININT = jnp.int32(-2147483648)
    CSHIFT = jnp.int32(1) - MININT
    i = pl.program_id(0)

    gx = -1.0 + jax.lax.broadcasted_iota(
        jnp.int32, (1, W), 1).astype(jnp.float32) * (2.0 / (W - 1))
    px = x_ref[:, 0:1]
    py = x_ref[:, 1:2]

    @pl.when(i == 0)
    def _():
        dxv = px - gx
        dx2_ref[...] = dxv * dxv

    dx2 = dx2_ref[...]
    row = jax.lax.broadcasted_iota(jnp.int32, (N, 1), 0)
    mask_hi = jnp.int32(~2047)

    for r, kref in ((0, key0_ref), (1, key1_ref)):
        gy = 1.0 - (2 * i + r).astype(jnp.float32) * (2.0 / (W - 1))
        dy = py - gy
        d2 = dx2 + dy * dy
        bits = jax.lax.bitcast_convert_type(d2, jnp.int32)
        kref[...] = (bits & mask_hi) | row

    def tree_min(e):
        while e.shape[0] > 1:
            h = e.shape[0] // 2
            e = jnp.minimum(e[:h], e[h:])
        return e

    def body(_, carry):
        s0, v0, s1, v1 = carry
        m0 = tree_min(key0_ref[...] - (v0 + CSHIFT))
        m1 = tree_min(key1_ref[...] - (v1 + CSHIFT))
        v0 = v0 + jnp.int32(1) + (m0 ^ MININT)
        v1 = v1 + jnp.int32(1) + (m1 ^ MININT)
        s0 = s0 + jax.lax.bitcast_convert_type(v0, jnp.float32)
        s1 = s1 + jax.lax.bitcast_convert_type(v1, jnp.float32)
        return s0, v0, s1, v1

    zf = jnp.zeros((1, W), jnp.float32)
    zi = jnp.full((1, W), -1, jnp.int32)
    s0, t0, s1, t1 = jax.lax.fori_loop(0, K, body, (zf, zi, zf, zi))

    for r, (s, t) in ((0, (s0, t0)), (1, (s1, t1))):
        tf = jax.lax.bitcast_convert_type(t, jnp.float32)
        out_ref[0, r:r + 1, :] = jnp.sqrt((s + tf * (BOUND - K)) / BOUND)


@jax.jit
def kernel(x):
    out = pl.pallas_call(
        _dtm_kernel,
        grid=(H // 2,),
        in_specs=[pl.BlockSpec((N, 2), lambda i: (0, 0))],
        out_specs=pl.BlockSpec((1, 2, W), lambda i: (i, 0, 0)),
        out_shape=jax.ShapeDtypeStruct((H // 2, 2, W), jnp.float32),
        scratch_shapes=[pltpu.VMEM((N, W), jnp.int32),
                        pltpu.VMEM((N, W), jnp.int32),
                        pltpu.VMEM((N, W), jnp.float32)],
    )(x)
    return out.reshape(H, W)


# speedup vs baseline: 4.7479x; 1.0119x over previous
import functools

import jax
import jax.numpy as jnp
from jax.experimental import pallas as pl
from jax.experimental.pallas import tpu as pltpu

N = 2048
H = 128
W = 128
M0 = 0.01
BOUND = M0 * N
K = 21


def _dtm_kernel(x_ref, out_ref, key0_ref, key1_ref, key2_ref, key3_ref,
                dx2_ref):
    MININT = jnp.int32(-2147483648)
    CSHIFT = jnp.int32(1) - MININT
    i = pl.program_id(0)

    gx = -1.0 + jax.lax.broadcasted_iota(
        jnp.int32, (1, W), 1).astype(jnp.float32) * (2.0 / (W - 1))
    px = x_ref[:, 0:1]
    py = x_ref[:, 1:2]

    @pl.when(i == 0)
    def _():
        dxv = px - gx
        dx2_ref[...] = dxv * dxv

    dx2 = dx2_ref[...]
    row = jax.lax.broadcasted_iota(jnp.int32, (N, 1), 0)
    mask_hi = jnp.int32(~2047)

    krefs = (key0_ref, key1_ref, key2_ref, key3_ref)
    for r, kref in enumerate(krefs):
        gy = 1.0 - (4 * i + r).astype(jnp.float32) * (2.0 / (W - 1))
        dy = py - gy
        d2 = dx2 + dy * dy
        bits = jax.lax.bitcast_convert_type(d2, jnp.int32)
        kref[...] = (bits & mask_hi) | row

    def tree_min(e):
        while e.shape[0] > 1:
            h = e.shape[0] // 2
            e = jnp.minimum(e[:h], e[h:])
        return e

    def body(_, carry):
        ss, vs = carry
        ms = [tree_min(kref[...] - (v + CSHIFT))
              for kref, v in zip(krefs, vs)]
        vs = tuple(v + jnp.int32(1) + (m ^ MININT) for v, m in zip(vs, ms))
        ss = tuple(s + jax.lax.bitcast_convert_type(v, jnp.float32)
                   for s, v in zip(ss, vs))
        return ss, vs

    zf = jnp.zeros((1, W), jnp.float32)
    zi = jnp.full((1, W), -1, jnp.int32)
    ss, ts = jax.lax.fori_loop(0, K, body, ((zf,) * 4, (zi,) * 4))

    for r in range(4):
        tf = jax.lax.bitcast_convert_type(ts[r], jnp.float32)
        out_ref[0, r:r + 1, :] = jnp.sqrt((ss[r] + tf * (BOUND - K)) / BOUND)


@jax.jit
def kernel(x):
    out = pl.pallas_call(
        _dtm_kernel,
        grid=(H // 4,),
        in_specs=[pl.BlockSpec((N, 2), lambda i: (0, 0))],
        out_specs=pl.BlockSpec((1, 4, W), lambda i: (i, 0, 0)),
        out_shape=jax.ShapeDtypeStruct((H // 4, 4, W), jnp.float32),
        scratch_shapes=[pltpu.VMEM((N, W), jnp.int32),
                        pltpu.VMEM((N, W), jnp.int32),
                        pltpu.VMEM((N, W), jnp.int32),
                        pltpu.VMEM((N, W), jnp.int32),
                        pltpu.VMEM((N, W), jnp.float32)],
    )(x)
    return out.reshape(H, W)


# speedup vs baseline: 5.7288x; 1.2066x over previous
import functools

import jax
import jax.numpy as jnp
from jax.experimental import pallas as pl
from jax.experimental.pallas import tpu as pltpu

N = 2048
H = 128
W = 128
M0 = 0.01
BOUND = M0 * N
K = 21
CH = 128
NR = 2


def _dtm_kernel(x_ref, out_ref, key0_ref, key1_ref, dx2_ref):
    MININT = jnp.int32(-2147483648)
    CSHIFT = jnp.int32(1) - MININT
    i = pl.program_id(0)

    gx = -1.0 + jax.lax.broadcasted_iota(
        jnp.int32, (1, W), 1).astype(jnp.float32) * (2.0 / (W - 1))
    px = x_ref[:, 0:1]
    py = x_ref[:, 1:2]

    @pl.when(i == 0)
    def _():
        dxv = px - gx
        dx2_ref[...] = dxv * dxv

    dx2 = dx2_ref[...]
    row = jax.lax.broadcasted_iota(jnp.int32, (N, 1), 0)
    mask_hi = jnp.int32(~2047)

    krefs = (key0_ref, key1_ref)
    for r, kref in enumerate(krefs):
        gy = 1.0 - (NR * i + r).astype(jnp.float32) * (2.0 / (W - 1))
        dy = py - gy
        d2 = dx2 + dy * dy
        bits = jax.lax.bitcast_convert_type(d2, jnp.int32)
        kref[...] = (bits & mask_hi) | row

    def masked_min(kref, shift):
        acc = None
        for c in range(0, N, CH):
            e = kref[c:c + CH, :] - shift
            while e.shape[0] > 8:
                h = e.shape[0] // 2
                e = jnp.minimum(e[:h], e[h:])
            acc = e if acc is None else jnp.minimum(acc, e)
        while acc.shape[0] > 1:
            h = acc.shape[0] // 2
            acc = jnp.minimum(acc[:h], acc[h:])
        return acc

    def body(_, carry):
        ss, vs = carry
        ms = [masked_min(kref, v + CSHIFT) for kref, v in zip(krefs, vs)]
        vs = tuple(v + jnp.int32(1) + (m ^ MININT) for v, m in zip(vs, ms))
        ss = tuple(s + jax.lax.bitcast_convert_type(v, jnp.float32)
                   for s, v in zip(ss, vs))
        return ss, vs

    zf = jnp.zeros((1, W), jnp.float32)
    zi = jnp.full((1, W), -1, jnp.int32)
    ss, ts = jax.lax.fori_loop(0, K, body, ((zf,) * NR, (zi,) * NR))

    for r in range(NR):
        tf = jax.lax.bitcast_convert_type(ts[r], jnp.float32)
        out_ref[0, r:r + 1, :] = jnp.sqrt((ss[r] + tf * (BOUND - K)) / BOUND)


@jax.jit
def kernel(x):
    out = pl.pallas_call(
        _dtm_kernel,
        grid=(H // NR,),
        in_specs=[pl.BlockSpec((N, 2), lambda i: (0, 0))],
        out_specs=pl.BlockSpec((1, NR, W), lambda i: (i, 0, 0)),
        out_shape=jax.ShapeDtypeStruct((H // NR, NR, W), jnp.float32),
        scratch_shapes=[pltpu.VMEM((N, W), jnp.int32),
                        pltpu.VMEM((N, W), jnp.int32),
                        pltpu.VMEM((N, W), jnp.float32)],
    )(x)
    return out.reshape(H, W)


# speedup vs baseline: 5.9564x; 1.0397x over previous
import functools

import jax
import jax.numpy as jnp
from jax.experimental import pallas as pl
from jax.experimental.pallas import tpu as pltpu

N = 2048
H = 128
W = 128
M0 = 0.01
BOUND = M0 * N
K = 21
CH = 128
NR = 4


def _dtm_kernel(x_ref, out_ref, *scratch):
    krefs, dx2_ref = scratch[:-1], scratch[-1]
    MININT = jnp.int32(-2147483648)
    CSHIFT = jnp.int32(1) - MININT
    i = pl.program_id(0)

    gx = -1.0 + jax.lax.broadcasted_iota(
        jnp.int32, (1, W), 1).astype(jnp.float32) * (2.0 / (W - 1))
    px = x_ref[:, 0:1]
    py = x_ref[:, 1:2]

    @pl.when(i == 0)
    def _():
        dxv = px - gx
        dx2_ref[...] = dxv * dxv

    dx2 = dx2_ref[...]
    row = jax.lax.broadcasted_iota(jnp.int32, (N, 1), 0)
    mask_hi = jnp.int32(~2047)

    for r, kref in enumerate(krefs):
        gy = 1.0 - (NR * i + r).astype(jnp.float32) * (2.0 / (W - 1))
        dy = py - gy
        d2 = dx2 + dy * dy
        bits = jax.lax.bitcast_convert_type(d2, jnp.int32)
        kref[...] = (bits & mask_hi) | row

    def masked_min(kref, shift):
        acc = None
        for c in range(0, N, CH):
            e = kref[c:c + CH, :] - shift
            while e.shape[0] > 8:
                h = e.shape[0] // 2
                e = jnp.minimum(e[:h], e[h:])
            acc = e if acc is None else jnp.minimum(acc, e)
        while acc.shape[0] > 1:
            h = acc.shape[0] // 2
            acc = jnp.minimum(acc[:h], acc[h:])
        return acc

    def body(_, carry):
        ss, vs = carry
        ms = [masked_min(kref, v + CSHIFT) for kref, v in zip(krefs, vs)]
        vs = tuple(v + jnp.int32(1) + (m ^ MININT) for v, m in zip(vs, ms))
        ss = tuple(s + jax.lax.bitcast_convert_type(v, jnp.float32)
                   for s, v in zip(ss, vs))
        return ss, vs

    zf = jnp.zeros((1, W), jnp.float32)
    zi = jnp.full((1, W), -1, jnp.int32)
    ss, ts = jax.lax.fori_loop(0, K, body, ((zf,) * NR, (zi,) * NR))

    for r in range(NR):
        tf = jax.lax.bitcast_convert_type(ts[r], jnp.float32)
        out_ref[0, r:r + 1, :] = jnp.sqrt((ss[r] + tf * (BOUND - K)) / BOUND)


@jax.jit
def kernel(x):
    out = pl.pallas_call(
        _dtm_kernel,
        grid=(H // NR,),
        in_specs=[pl.BlockSpec((N, 2), lambda i: (0, 0))],
        out_specs=pl.BlockSpec((1, NR, W), lambda i: (i, 0, 0)),
        out_shape=jax.ShapeDtypeStruct((H // NR, NR, W), jnp.float32),
        scratch_shapes=[pltpu.VMEM((N, W), jnp.int32)] * NR
                       + [pltpu.VMEM((N, W), jnp.float32)],
    )(x)
    return out.reshape(H, W)


# speedup vs baseline: 6.1170x; 1.0270x over previous
import functools

import jax
import jax.numpy as jnp
from jax.experimental import pallas as pl
from jax.experimental.pallas import tpu as pltpu

N = 2048
H = 128
W = 128
M0 = 0.01
BOUND = M0 * N
K = 21
CH = 128
NR = 8


def _dtm_kernel(x_ref, out_ref, *scratch):
    krefs, dx2_ref = scratch[:-1], scratch[-1]
    MININT = jnp.int32(-2147483648)
    CSHIFT = jnp.int32(1) - MININT
    i = pl.program_id(0)

    gx = -1.0 + jax.lax.broadcasted_iota(
        jnp.int32, (1, W), 1).astype(jnp.float32) * (2.0 / (W - 1))
    px = x_ref[:, 0:1]
    py = x_ref[:, 1:2]

    @pl.when(i == 0)
    def _():
        dxv = px - gx
        dx2_ref[...] = dxv * dxv

    dx2 = dx2_ref[...]
    row = jax.lax.broadcasted_iota(jnp.int32, (N, 1), 0)
    mask_hi = jnp.int32(~2047)

    for r, kref in enumerate(krefs):
        gy = 1.0 - (NR * i + r).astype(jnp.float32) * (2.0 / (W - 1))
        dy = py - gy
        d2 = dx2 + dy * dy
        bits = jax.lax.bitcast_convert_type(d2, jnp.int32)
        kref[...] = (bits & mask_hi) | row

    def masked_min(kref, shift):
        acc = None
        for c in range(0, N, CH):
            e = kref[c:c + CH, :] - shift
            while e.shape[0] > 8:
                h = e.shape[0] // 2
                e = jnp.minimum(e[:h], e[h:])
            acc = e if acc is None else jnp.minimum(acc, e)
        while acc.shape[0] > 1:
            h = acc.shape[0] // 2
            acc = jnp.minimum(acc[:h], acc[h:])
        return acc

    def body(_, carry):
        ss, vs = carry
        ms = [masked_min(kref, v + CSHIFT) for kref, v in zip(krefs, vs)]
        vs = tuple(v + jnp.int32(1) + (m ^ MININT) for v, m in zip(vs, ms))
        ss = tuple(s + jax.lax.bitcast_convert_type(v, jnp.float32)
                   for s, v in zip(ss, vs))
        return ss, vs

    zf = jnp.zeros((1, W), jnp.float32)
    zi = jnp.full((1, W), -1, jnp.int32)
    ss, ts = jax.lax.fori_loop(0, K, body, ((zf,) * NR, (zi,) * NR))

    for r in range(NR):
        tf = jax.lax.bitcast_convert_type(ts[r], jnp.float32)
        out_ref[0, r:r + 1, :] = jnp.sqrt((ss[r] + tf * (BOUND - K)) / BOUND)


@jax.jit
def kernel(x):
    out = pl.pallas_call(
        _dtm_kernel,
        grid=(H // NR,),
        in_specs=[pl.BlockSpec((N, 2), lambda i: (0, 0))],
        out_specs=pl.BlockSpec((1, NR, W), lambda i: (i, 0, 0)),
        out_shape=jax.ShapeDtypeStruct((H // NR, NR, W), jnp.float32),
        scratch_shapes=[pltpu.VMEM((N, W), jnp.int32)] * NR
                       + [pltpu.VMEM((N, W), jnp.float32)],
    )(x)
    return out.reshape(H, W)
